# baseline (speedup 1.0000x reference)
import functools

import jax
import jax.numpy as jnp
from jax import lax
from jax.experimental import pallas as pl
from jax.experimental.pallas import tpu as pltpu
from jax.experimental.pallas import tpu_sc as plsc

N_NODES = 10000
N_EDGES = 320000
D = 128
N_GRAPHS = 1024
BATCH = 128
N_LEMMAS = 1000

NC = 2
NS = 16
NPAD = 10112
STRIPE = NPAD // NS
EPW = N_EDGES // (NC * NS)
CH = 128
NCHUNK = EPW // CH
TAIL = EPW - NCHUNK * CH
NPAD_G = 10240
GPW = NPAD_G // (NC * NS)
GCHUNKS = ((0, 128), (128, 128), (256, 64))


def _sc_edge_agg(x, src2, dst2, zrows, bg, gnnp):
    mesh = plsc.VectorSubcoreMesh(core_axis_name="c", subcore_axis_name="s")

    @functools.partial(
        pl.kernel,
        mesh=mesh,
        out_type=[
            jax.ShapeDtypeStruct((NC, NPAD, D), jnp.float32),
            jax.ShapeDtypeStruct((NC * NS, GPW), jnp.int32),
        ],
        scratch_types=[
            pltpu.VMEM((CH,), jnp.int32),
            pltpu.VMEM((CH,), jnp.int32),
            pltpu.VMEM((CH,), jnp.int32),
            pltpu.VMEM((CH,), jnp.int32),
            pltpu.VMEM((TAIL,), jnp.int32),
            pltpu.VMEM((TAIL,), jnp.int32),
            pltpu.VMEM((CH, D), jnp.float32),
            pltpu.VMEM((CH, D), jnp.float32),
            pltpu.VMEM((TAIL, D), jnp.float32),
            pltpu.VMEM((GPW,), jnp.int32),
            pltpu.VMEM((GPW,), jnp.int32),
            pltpu.VMEM_SHARED((NPAD, D), jnp.float32),
            pltpu.SemaphoreType.DMA,
            pltpu.SemaphoreType.DMA,
            pltpu.SemaphoreType.DMA,
            pltpu.SemaphoreType.DMA,
            pltpu.SemaphoreType.DMA,
            pltpu.SemaphoreType.DMA,
            pltpu.SemaphoreType.DMA,
        ],
    )
    def k(x_hbm, src_hbm, dst_hbm, z_hbm, bg_hbm, gnn_hbm, out_hbm, obj_hbm,
          srcv0, dstv0, srcv1, dstv1, srcvt, dstvt, rows0, rows1, rowst,
          gl, ol, acc, semg0, semg1, semi0, semi1, sems0, sems1, semo):
        cid = lax.axis_index("c")
        sid = lax.axis_index("s")
        wid = cid * NS + sid

        pltpu.sync_copy(gnn_hbm.at[wid], gl)
        for (o, n) in GCHUNKS:
            pltpu.async_copy(bg_hbm.at[gl.at[pl.ds(o, n)]],
                             ol.at[pl.ds(o, n)], semo)

        pltpu.sync_copy(z_hbm, acc.at[pl.ds(sid * STRIPE, STRIPE)])
        plsc.subcore_barrier()

        def idx_start(j, sv, dv, sem):
            pltpu.async_copy(src_hbm.at[wid, pl.ds(j * CH, CH)], sv, sem)
            pltpu.async_copy(dst_hbm.at[wid, pl.ds(j * CH, CH)], dv, sem)

        def idx_wait(j, sv, dv, sem):
            pltpu.make_async_copy(src_hbm.at[wid, pl.ds(j * CH, CH)], sv, sem).wait()
            pltpu.make_async_copy(dst_hbm.at[wid, pl.ds(j * CH, CH)], dv, sem).wait()

        def sc_start(rows, dv, sem):
            pltpu.async_copy(rows, acc.at[dv], sem, add=True)

        def sc_wait(rows, dv, sem):
            pltpu.make_async_copy(rows, acc.at[dv], sem).wait()

        idx_start(0, srcv0, dstv0, semi0)
        idx_start(1, srcv1, dstv1, semi1)
        idx_wait(0, srcv0, dstv0, semi0)
        pltpu.async_copy(x_hbm.at[srcv0], rows0, semg0)
        idx_wait(1, srcv1, dstv1, semi1)

        def body(it, carry):
            j0 = it * 2
            pltpu.make_async_copy(x_hbm.at[srcv0], rows0, semg0).wait()
            sc_start(rows0, dstv0, sems0)

            @pl.when(it > 0)
            def _():
                sc_wait(rows1, dstv1, sems1)

            pltpu.async_copy(x_hbm.at[srcv1], rows1, semg1)
            pltpu.make_async_copy(x_hbm.at[srcv1], rows1, semg1).wait()
            sc_start(rows1, dstv1, sems1)
            sc_wait(rows0, dstv0, sems0)

            @pl.when(j0 + 2 < NCHUNK)
            def _():
                idx_start(j0 + 2, srcv0, dstv0, semi0)
                idx_wait(j0 + 2, srcv0, dstv0, semi0)
                pltpu.async_copy(x_hbm.at[srcv0], rows0, semg0)

            @pl.when(j0 + 3 < NCHUNK)
            def _():
                idx_start(j0 + 3, srcv1, dstv1, semi1)
                idx_wait(j0 + 3, srcv1, dstv1, semi1)

            return carry

        lax.fori_loop(0, NCHUNK // 2, body, 0)
        sc_wait(rows1, dstv1, sems1)

        tb = NCHUNK * CH
        pltpu.sync_copy(src_hbm.at[wid, pl.ds(tb, TAIL)], srcvt)
        pltpu.sync_copy(dst_hbm.at[wid, pl.ds(tb, TAIL)], dstvt)
        pltpu.async_copy(x_hbm.at[srcvt], rowst, semg0).wait()
        pltpu.sync_copy(rowst, acc.at[dstvt], add=True)

        for (o, n) in GCHUNKS:
            pltpu.make_async_copy(bg_hbm.at[gl.at[pl.ds(o, n)]],
                                  ol.at[pl.ds(o, n)], semo).wait()
        pltpu.sync_copy(ol, obj_hbm.at[wid])

        plsc.subcore_barrier()
        pltpu.sync_copy(acc.at[pl.ds(sid * STRIPE, STRIPE)],
                        out_hbm.at[cid, pl.ds(sid * STRIPE, STRIPE)])

    return k(x, src2, dst2, zrows, bg, gnnp)


NBLK = 10
BLK = N_NODES // NBLK


def _tc_body(pref, xref, oref, wmsg, wself,
             wv1, bv1, wv2, bv2, wq1, bq1, wq2, bq2, wl1, wl2, bl,
             vf_ref, log_ref, oacc):
    i = pl.program_id(0)

    @pl.when(i == 0)
    def _():
        oacc[...] = jnp.zeros_like(oacc)

    xa = pref[0] + pref[1]
    state = jnp.maximum(
        jnp.dot(xa, wmsg[...], preferred_element_type=jnp.float32)
        + jnp.dot(xref[...], wself[...], preferred_element_type=jnp.float32),
        0.0)
    g = oref[0]
    oh = (g == lax.broadcasted_iota(jnp.int32, (BATCH, BLK), 0)
          ).astype(jnp.float32)
    oacc[...] += jnp.dot(oh, state, preferred_element_type=jnp.float32)

    @pl.when(i == NBLK - 1)
    def _():
        obj = oacc[...]
        v = jnp.maximum(
            jnp.dot(obj, wv1[...], preferred_element_type=jnp.float32)
            + bv1[...], 0.0)
        vf_ref[...] = jax.nn.sigmoid(
            jnp.dot(v, wv2[...], preferred_element_type=jnp.float32)
            + bv2[...])
        h = jnp.dot(
            jnp.maximum(
                jnp.dot(obj, wq1[...], preferred_element_type=jnp.float32)
                + bq1[...], 0.0),
            wq2[...], preferred_element_type=jnp.float32) + bq2[...]
        q1 = jnp.maximum(obj + h[:, :D], 0.0)
        q2 = jnp.maximum(h[:, D:], 0.0)
        log_ref[...] = (
            jnp.dot(q1, wl1[...], preferred_element_type=jnp.float32)
            + jnp.dot(q2, wl2[...], preferred_element_type=jnp.float32)
            + bl[...])


def kernel(x, edge_index, gnn_ind, batch_gnn_ind, W_msg, W_self,
           Wq1, bq1, Wq2, bq2, Wl, bl, Wv1, bv1, Wv2, bv2):
    src = edge_index[0].astype(jnp.int32)
    dst = edge_index[1].astype(jnp.int32)
    src2 = src.reshape(NC * NS, EPW)
    dst2 = dst.reshape(NC * NS, EPW)
    zrows = jnp.zeros((STRIPE, D), jnp.float32)
    gi = gnn_ind.astype(jnp.int32)
    gnnp = jnp.concatenate(
        [gi, jnp.zeros((NPAD_G - N_NODES,), jnp.int32)]).reshape(NC * NS, GPW)
    bg = batch_gnn_ind.astype(jnp.int32)

    p, obj_ind = _sc_edge_agg(x, src2, dst2, zrows, bg, gnnp)

    obj3 = obj_ind.reshape(-1)[:N_NODES].reshape(NBLK, 1, BLK)

    full = lambda s: pl.BlockSpec(s, lambda i: tuple(0 for _ in s))
    vf, logits = pl.pallas_call(
        _tc_body,
        grid=(NBLK,),
        in_specs=[
            pl.BlockSpec((NC, BLK, D), lambda i: (0, i, 0)),
            pl.BlockSpec((BLK, D), lambda i: (i, 0)),
            pl.BlockSpec((1, 1, BLK), lambda i: (i, 0, 0)),
            full((D, D)), full((D, D)),
            full((D, D)), full((1, D)), full((D, 1)), full((1, 1)),
            full((D, 2 * D)), full((1, 2 * D)),
            full((2 * D, 2 * D)), full((1, 2 * D)),
            full((D, N_LEMMAS)), full((D, N_LEMMAS)), full((1, N_LEMMAS)),
        ],
        out_specs=[
            pl.BlockSpec((BATCH, 1), lambda i: (0, 0)),
            pl.BlockSpec((BATCH, N_LEMMAS), lambda i: (0, 0)),
        ],
        out_shape=[
            jax.ShapeDtypeStruct((BATCH, 1), jnp.float32),
            jax.ShapeDtypeStruct((BATCH, N_LEMMAS), jnp.float32),
        ],
        scratch_shapes=[pltpu.VMEM((BATCH, D), jnp.float32)],
    )(p, x, obj3, W_msg, W_self,
      Wv1[:D], bv1.reshape(1, D), Wv2, bv2.reshape(1, 1),
      Wq1[:D], bq1.reshape(1, 2 * D), Wq2, bq2.reshape(1, 2 * D),
      Wl[:D], Wl[D:], bl.reshape(1, N_LEMMAS))

    return jnp.concatenate([vf, logits], axis=1)


# speedup vs baseline: 1.1342x; 1.1342x over previous
import functools

import jax
import jax.numpy as jnp
from jax import lax
from jax.experimental import pallas as pl
from jax.experimental.pallas import tpu as pltpu
from jax.experimental.pallas import tpu_sc as plsc

N_NODES = 10000
N_EDGES = 320000
D = 128
N_GRAPHS = 1024
BATCH = 128
N_LEMMAS = 1000

NC = 2
NS = 16
NPAD = 10112
STRIPE = NPAD // NS
EPW = N_EDGES // (NC * NS)
CH = 128
NCHUNK = EPW // CH
TAIL = EPW - NCHUNK * CH
NPAD_G = 10240
GPW = NPAD_G // (NC * NS)
GCHUNKS = ((0, 128), (128, 128), (256, 64))


def _sc_edge_agg(x, src2, dst2, zrows, bg, gnnp):
    mesh = plsc.VectorSubcoreMesh(core_axis_name="c", subcore_axis_name="s")

    @functools.partial(
        pl.kernel,
        mesh=mesh,
        out_type=[
            jax.ShapeDtypeStruct((NC, NPAD, D), jnp.float32),
            jax.ShapeDtypeStruct((NC * NS, GPW), jnp.int32),
        ],
        scratch_types=[
            pltpu.VMEM((CH,), jnp.int32),
            pltpu.VMEM((CH,), jnp.int32),
            pltpu.VMEM((CH,), jnp.int32),
            pltpu.VMEM((CH,), jnp.int32),
            pltpu.VMEM((TAIL,), jnp.int32),
            pltpu.VMEM((TAIL,), jnp.int32),
            pltpu.VMEM((CH, D), jnp.float32),
            pltpu.VMEM((CH, D), jnp.float32),
            pltpu.VMEM((TAIL, D), jnp.float32),
            pltpu.VMEM((GPW,), jnp.int32),
            pltpu.VMEM((GPW,), jnp.int32),
            pltpu.VMEM_SHARED((NPAD, D), jnp.float32),
            pltpu.SemaphoreType.DMA,
            pltpu.SemaphoreType.DMA,
            pltpu.SemaphoreType.DMA,
            pltpu.SemaphoreType.DMA,
            pltpu.SemaphoreType.DMA,
        ],
    )
    def k(x_hbm, src_hbm, dst_hbm, z_hbm, bg_hbm, gnn_hbm, out_hbm, obj_hbm,
          srcv0, dstv0, srcv1, dstv1, srcvt, dstvt, rows0, rows1, rowst,
          gl, ol, acc, semg0, semg1, semi0, semi1, semo):
        cid = lax.axis_index("c")
        sid = lax.axis_index("s")
        wid = cid * NS + sid

        pltpu.sync_copy(gnn_hbm.at[wid], gl)
        for (o, n) in GCHUNKS:
            pltpu.async_copy(bg_hbm.at[gl.at[pl.ds(o, n)]],
                             ol.at[pl.ds(o, n)], semo)

        pltpu.sync_copy(z_hbm, acc.at[pl.ds(sid * STRIPE, STRIPE)])
        plsc.subcore_barrier()

        def idx_start(j, sv, dv, sem):
            pltpu.async_copy(src_hbm.at[wid, pl.ds(j * CH, CH)], sv, sem)
            pltpu.async_copy(dst_hbm.at[wid, pl.ds(j * CH, CH)], dv, sem)

        def idx_wait(j, sv, dv, sem):
            pltpu.make_async_copy(src_hbm.at[wid, pl.ds(j * CH, CH)], sv, sem).wait()
            pltpu.make_async_copy(dst_hbm.at[wid, pl.ds(j * CH, CH)], dv, sem).wait()

        idx_start(0, srcv0, dstv0, semi0)
        idx_start(1, srcv1, dstv1, semi1)
        idx_wait(0, srcv0, dstv0, semi0)
        pltpu.async_copy(x_hbm.at[srcv0], rows0, semg0)
        idx_wait(1, srcv1, dstv1, semi1)

        def body(it, carry):
            j0 = it * 2
            pltpu.async_copy(x_hbm.at[srcv1], rows1, semg1)
            pltpu.make_async_copy(x_hbm.at[srcv0], rows0, semg0).wait()
            pltpu.sync_copy(rows0, acc.at[dstv0], add=True)

            @pl.when(j0 + 2 < NCHUNK)
            def _():
                idx_start(j0 + 2, srcv0, dstv0, semi0)
                idx_wait(j0 + 2, srcv0, dstv0, semi0)
                pltpu.async_copy(x_hbm.at[srcv0], rows0, semg0)

            pltpu.make_async_copy(x_hbm.at[srcv1], rows1, semg1).wait()
            pltpu.sync_copy(rows1, acc.at[dstv1], add=True)

            @pl.when(j0 + 3 < NCHUNK)
            def _():
                idx_start(j0 + 3, srcv1, dstv1, semi1)
                idx_wait(j0 + 3, srcv1, dstv1, semi1)

            return carry

        lax.fori_loop(0, NCHUNK // 2, body, 0)

        tb = NCHUNK * CH
        pltpu.sync_copy(src_hbm.at[wid, pl.ds(tb, TAIL)], srcvt)
        pltpu.sync_copy(dst_hbm.at[wid, pl.ds(tb, TAIL)], dstvt)
        pltpu.async_copy(x_hbm.at[srcvt], rowst, semg0).wait()
        pltpu.sync_copy(rowst, acc.at[dstvt], add=True)

        for (o, n) in GCHUNKS:
            pltpu.make_async_copy(bg_hbm.at[gl.at[pl.ds(o, n)]],
                                  ol.at[pl.ds(o, n)], semo).wait()
        pltpu.sync_copy(ol, obj_hbm.at[wid])

        plsc.subcore_barrier()
        pltpu.sync_copy(acc.at[pl.ds(sid * STRIPE, STRIPE)],
                        out_hbm.at[cid, pl.ds(sid * STRIPE, STRIPE)])

    return k(x, src2, dst2, zrows, bg, gnnp)


NBLK = 10
BLK = N_NODES // NBLK


def _tc_body(pref, xref, oref, wmsg, wself,
             wv1, bv1, wv2, bv2, wq1, bq1, wq2, bq2, wl1, wl2, bl,
             vf_ref, log_ref, oacc):
    i = pl.program_id(0)

    @pl.when(i == 0)
    def _():
        oacc[...] = jnp.zeros_like(oacc)

    xa = pref[0] + pref[1]
    state = jnp.maximum(
        jnp.dot(xa, wmsg[...], preferred_element_type=jnp.float32)
        + jnp.dot(xref[...], wself[...], preferred_element_type=jnp.float32),
        0.0)
    g = oref[0]
    oh = (g == lax.broadcasted_iota(jnp.int32, (BATCH, BLK), 0)
          ).astype(jnp.float32)
    oacc[...] += jnp.dot(oh, state, preferred_element_type=jnp.float32)

    @pl.when(i == NBLK - 1)
    def _():
        obj = oacc[...]
        v = jnp.maximum(
            jnp.dot(obj, wv1[...], preferred_element_type=jnp.float32)
            + bv1[...], 0.0)
        vf_ref[...] = jax.nn.sigmoid(
            jnp.dot(v, wv2[...], preferred_element_type=jnp.float32)
            + bv2[...])
        h = jnp.dot(
            jnp.maximum(
                jnp.dot(obj, wq1[...], preferred_element_type=jnp.float32)
                + bq1[...], 0.0),
            wq2[...], preferred_element_type=jnp.float32) + bq2[...]
        q1 = jnp.maximum(obj + h[:, :D], 0.0)
        q2 = jnp.maximum(h[:, D:], 0.0)
        log_ref[...] = (
            jnp.dot(q1, wl1[...], preferred_element_type=jnp.float32)
            + jnp.dot(q2, wl2[...], preferred_element_type=jnp.float32)
            + bl[...])


def kernel(x, edge_index, gnn_ind, batch_gnn_ind, W_msg, W_self,
           Wq1, bq1, Wq2, bq2, Wl, bl, Wv1, bv1, Wv2, bv2):
    src = edge_index[0].astype(jnp.int32)
    dst = edge_index[1].astype(jnp.int32)
    src2 = src.reshape(NC * NS, EPW)
    dst2 = dst.reshape(NC * NS, EPW)
    zrows = jnp.zeros((STRIPE, D), jnp.float32)
    gi = gnn_ind.astype(jnp.int32)
    gnnp = jnp.concatenate(
        [gi, jnp.zeros((NPAD_G - N_NODES,), jnp.int32)]).reshape(NC * NS, GPW)
    bg = batch_gnn_ind.astype(jnp.int32)

    p, obj_ind = _sc_edge_agg(x, src2, dst2, zrows, bg, gnnp)

    obj3 = obj_ind.reshape(-1)[:N_NODES].reshape(NBLK, 1, BLK)

    full = lambda s: pl.BlockSpec(s, lambda i: tuple(0 for _ in s))
    vf, logits = pl.pallas_call(
        _tc_body,
        grid=(NBLK,),
        in_specs=[
            pl.BlockSpec((NC, BLK, D), lambda i: (0, i, 0)),
            pl.BlockSpec((BLK, D), lambda i: (i, 0)),
            pl.BlockSpec((1, 1, BLK), lambda i: (i, 0, 0)),
            full((D, D)), full((D, D)),
            full((D, D)), full((1, D)), full((D, 1)), full((1, 1)),
            full((D, 2 * D)), full((1, 2 * D)),
            full((2 * D, 2 * D)), full((1, 2 * D)),
            full((D, N_LEMMAS)), full((D, N_LEMMAS)), full((1, N_LEMMAS)),
        ],
        out_specs=[
            pl.BlockSpec((BATCH, 1), lambda i: (0, 0)),
            pl.BlockSpec((BATCH, N_LEMMAS), lambda i: (0, 0)),
        ],
        out_shape=[
            jax.ShapeDtypeStruct((BATCH, 1), jnp.float32),
            jax.ShapeDtypeStruct((BATCH, N_LEMMAS), jnp.float32),
        ],
        scratch_shapes=[pltpu.VMEM((BATCH, D), jnp.float32)],
    )(p, x, obj3, W_msg, W_self,
      Wv1[:D], bv1.reshape(1, D), Wv2, bv2.reshape(1, 1),
      Wq1[:D], bq1.reshape(1, 2 * D), Wq2, bq2.reshape(1, 2 * D),
      Wl[:D], Wl[D:], bl.reshape(1, N_LEMMAS))

    return jnp.concatenate([vf, logits], axis=1)
